# baseline (speedup 1.0000x reference)
import functools

import jax
import jax.numpy as jnp
from jax import lax
from jax.experimental import pallas as pl
from jax.experimental.pallas import tpu as pltpu
from jax.experimental.pallas import tpu_sc as plsc

_NC = 2
_NS = 16
_G = 32
_CW = 16


def _largest_divisor(n, cap):
    for c in range(cap, 0, -1):
        if n % c == 0 and c % 8 == 0:
            return c
    return None


def _sc_aggregate(y, src, dst):
    n, w = y.shape
    e = src.shape[0]
    nw = _NC * _NS
    assert e % nw == 0 and n % _NS == 0
    ept = e // nw
    ch = _largest_divisor(ept, 128)
    nchunk = ept // ch
    rows_pt = n // _NS
    nbuf = 3

    mesh = plsc.VectorSubcoreMesh(core_axis_name="c", subcore_axis_name="s")
    out_type = jax.ShapeDtypeStruct((_NC, _NS, rows_pt, w), jnp.float32)
    assert nchunk >= nbuf + 1
    scratch = (
        [pltpu.VMEM((ept,), jnp.int32),
         pltpu.VMEM((ept,), jnp.int32)]
        + [pltpu.VMEM((ch, w), jnp.float32) for _ in range(nbuf)]
        + [pltpu.VMEM_SHARED((n, w), jnp.float32)]
        + [pltpu.SemaphoreType.DMA for _ in range(nbuf)]
        + [pltpu.SemaphoreType.DMA for _ in range(nbuf)]
        + [pltpu.SemaphoreType.DMA,
           pltpu.SemaphoreType.DMA]
    )

    def body(y_hbm, src_hbm, dst_hbm, zer_hbm, acc_out, *rest):
        srcs_v, dsts_v = rest[:2]
        bufs = rest[2:2 + nbuf]
        acc_s = rest[2 + nbuf]
        semgs = rest[3 + nbuf:3 + 2 * nbuf]
        semss = rest[3 + 2 * nbuf:3 + 3 * nbuf]
        semi, semz = rest[3 + 3 * nbuf:5 + 3 * nbuf]
        cid = lax.axis_index("c")
        sid = lax.axis_index("s")
        wid = sid * _NC + cid

        ebase = wid * ept
        pltpu.async_copy(src_hbm.at[pl.ds(ebase, ept)], srcs_v, semi)
        pltpu.async_copy(dst_hbm.at[pl.ds(ebase, ept)], dsts_v, semi)

        r0 = sid * rows_pt
        zsrc = zer_hbm.at[sid]
        zdst = acc_s.at[pl.ds(r0, rows_pt)]
        pltpu.async_copy(zsrc, zdst, semz)
        pltpu.make_async_copy(zsrc, zdst, semz).wait()
        plsc.subcore_barrier()

        pltpu.make_async_copy(src_hbm.at[pl.ds(ebase, ept)], srcs_v,
                              semi).wait()
        pltpu.make_async_copy(dst_hbm.at[pl.ds(ebase, ept)], dsts_v,
                              semi).wait()

        def gath(j, buf, sem):
            pltpu.async_copy(y_hbm.at[srcs_v.at[pl.ds(j * ch, ch)]],
                             buf, sem)

        def gwait(j, buf, sem):
            pltpu.make_async_copy(y_hbm.at[srcs_v.at[pl.ds(j * ch, ch)]],
                                  buf, sem).wait()

        def sstart(j, buf, sem):
            pltpu.async_copy(buf, acc_s.at[dsts_v.at[pl.ds(j * ch, ch)]],
                             sem, add=True)

        def swait(j, buf, sem):
            pltpu.make_async_copy(buf,
                                  acc_s.at[dsts_v.at[pl.ds(j * ch, ch)]],
                                  sem).wait()

        gath(0, bufs[0], semgs[0])
        for j in range(min(2, nchunk)):
            if j + 1 < nchunk:
                gath(j + 1, bufs[(j + 1) % nbuf], semgs[(j + 1) % nbuf])
            gwait(j, bufs[j % nbuf], semgs[j % nbuf])
            sstart(j, bufs[j % nbuf], semss[j % nbuf])

        def group(k, _):
            for i in range(nbuf):
                j = 2 + nbuf * k + i
                sl = (2 + i) % nbuf
                sn = (2 + i + 1) % nbuf
                swait(j - 2, bufs[sn], semss[sn])
                gath(j + 1, bufs[sn], semgs[sn])
                gwait(j, bufs[sl], semgs[sl])
                sstart(j, bufs[sl], semss[sl])
            return 0

        ngrp = max((nchunk - 3) // nbuf, 0)
        lax.fori_loop(0, ngrp, group, 0)
        for j in range(2 + nbuf * ngrp, nchunk):
            sl = j % nbuf
            sn = (j + 1) % nbuf
            if j + 1 < nchunk:
                swait(j - 2, bufs[sn], semss[sn])
                gath(j + 1, bufs[sn], semgs[sn])
            gwait(j, bufs[sl], semgs[sl])
            sstart(j, bufs[sl], semss[sl])
        for j in range(max(nchunk - 3, 0), nchunk):
            swait(j, bufs[j % nbuf], semss[j % nbuf])
        plsc.subcore_barrier()

        pltpu.sync_copy(acc_s.at[pl.ds(r0, rows_pt)], acc_out.at[cid, sid])

    fn = pl.kernel(body, out_type=out_type, mesh=mesh, scratch_types=scratch)
    res = fn(y, src, dst, jnp.zeros((_NS, rows_pt, w), jnp.float32))
    return res.reshape(_NC, n, w)


def _sc_count(dst, n):
    e = dst.shape[0]
    nw = _NC * _NS
    assert e % nw == 0
    ept = e // nw
    ch = _largest_divisor(ept, 128)
    nchunk = ept // ch
    cpt = -(-n // (8 * _NS)) * 8
    ncpad = cpt * _NS
    zcn = -(-cpt // 16) * 16

    mesh = plsc.VectorSubcoreMesh(core_axis_name="c", subcore_axis_name="s")
    out_type = jax.ShapeDtypeStruct((_NC * ncpad,), jnp.float32)
    scratch = [
        pltpu.VMEM((ept,), jnp.int32),
        pltpu.VMEM((ch,), jnp.float32),
        pltpu.VMEM((zcn,), jnp.float32),
        pltpu.VMEM_SHARED((ncpad,), jnp.float32),
        pltpu.SemaphoreType.DMA,
    ]

    def body(dst_hbm, cnt_out, dsts_v, ones_v, zc_v, cnt_s, semi):
        cid = lax.axis_index("c")
        sid = lax.axis_index("s")
        wid = sid * _NC + cid
        ebase = wid * ept
        pltpu.async_copy(dst_hbm.at[pl.ds(ebase, ept)], dsts_v, semi)

        def fill16(ref, val):
            def go(j, _):
                ref[pl.ds(j * 16, 16)] = jnp.full((16,), val, jnp.float32)
                return 0
            lax.fori_loop(0, ref.shape[0] // 16, go, 0)

        fill16(ones_v, 1.0)
        fill16(zc_v, 0.0)
        pltpu.sync_copy(zc_v.at[pl.ds(0, cpt)],
                        cnt_s.at[pl.ds(sid * cpt, cpt)])
        pltpu.make_async_copy(dst_hbm.at[pl.ds(ebase, ept)], dsts_v,
                              semi).wait()
        plsc.subcore_barrier()

        def step(j, _):
            pltpu.sync_copy(ones_v,
                            cnt_s.at[dsts_v.at[pl.ds(j * ch, ch)]],
                            add=True)
            return 0

        lax.fori_loop(0, nchunk, step, 0)
        plsc.subcore_barrier()
        pltpu.sync_copy(cnt_s.at[pl.ds(sid * cpt, cpt)],
                        zc_v.at[pl.ds(0, cpt)])
        pltpu.sync_copy(zc_v.at[pl.ds(0, cpt)],
                        cnt_out.at[pl.ds(cid * ncpad + sid * cpt, cpt)])

    fn = pl.kernel(body, out_type=out_type, mesh=mesh, scratch_types=scratch)
    return fn(dst).reshape(_NC, ncpad)[:, :, None]


def _dot(a, b):
    return jnp.dot(a, b, preferred_element_type=jnp.float32)


def _mean_from_partials(p_ref, c_ref):
    psum = p_ref[0] + p_ref[1]
    cnt = jnp.maximum(c_ref[0] + c_ref[1], 1.0)
    return psum / cnt


def _tc_layer1(agg, cntp, x, W_l1, b_l1, W_r1, W_l2, b_l2, W_r2, interpret=False):
    n, d = x.shape
    k1 = W_l1.shape[1]
    k2 = W_l2.shape[1]
    R = 1000
    grid = (n // R,)

    def body(p_ref, c_ref, x_ref, wl1, bl1, wr1, wl2, bl2, wr2, y2_ref, s2_ref):
        mean = _mean_from_partials(p_ref, c_ref)
        h1 = jnp.maximum(
            _dot(mean, wl1[...]) + bl1[...] + _dot(x_ref[...], wr1[...]), 0.0)
        y2_ref[...] = _dot(h1, wl2[...])
        s2_ref[...] = _dot(h1, wr2[...]) + bl2[...]

    return pl.pallas_call(
        body,
        grid=grid,
        in_specs=[
            pl.BlockSpec((_NC, R, d), lambda i: (0, i, 0)),
            pl.BlockSpec((_NC, R, 1), lambda i: (0, i, 0)),
            pl.BlockSpec((R, d), lambda i: (i, 0)),
            pl.BlockSpec((d, k1), lambda i: (0, 0)),
            pl.BlockSpec((1, k1), lambda i: (0, 0)),
            pl.BlockSpec((d, k1), lambda i: (0, 0)),
            pl.BlockSpec((k1, k2), lambda i: (0, 0)),
            pl.BlockSpec((1, k2), lambda i: (0, 0)),
            pl.BlockSpec((k1, k2), lambda i: (0, 0)),
        ],
        out_specs=[
            pl.BlockSpec((R, k2), lambda i: (i, 0)),
            pl.BlockSpec((R, k2), lambda i: (i, 0)),
        ],
        out_shape=[
            jax.ShapeDtypeStruct((n, k2), jnp.float32),
            jax.ShapeDtypeStruct((n, k2), jnp.float32),
        ],
        interpret=interpret,
    )(agg, cntp, x, W_l1, b_l1.reshape(1, -1), W_r1, W_l2,
      b_l2.reshape(1, -1), W_r2)


def _tc_layer2(agg, cntp, s2, Wcat, bcat, interpret=False):
    n, d = s2.shape
    k2 = Wcat.shape[1]
    R = 1000
    grid = (n // R,)

    def body(p_ref, c_ref, s2_ref, wcat, bc, ycat_ref):
        mean = _mean_from_partials(p_ref, c_ref)
        h2 = jnp.maximum(mean + s2_ref[...], 0.0)
        ycat_ref[...] = _dot(h2, wcat[...]) + bc[...]

    return pl.pallas_call(
        body,
        grid=grid,
        in_specs=[
            pl.BlockSpec((_NC, R, d), lambda i: (0, i, 0)),
            pl.BlockSpec((_NC, R, 1), lambda i: (0, i, 0)),
            pl.BlockSpec((R, d), lambda i: (i, 0)),
            pl.BlockSpec((d, k2), lambda i: (0, 0)),
            pl.BlockSpec((1, k2), lambda i: (0, 0)),
        ],
        out_specs=pl.BlockSpec((R, k2), lambda i: (i, 0)),
        out_shape=jax.ShapeDtypeStruct((n, k2), jnp.float32),
        interpret=interpret,
    )(agg, cntp, s2, Wcat, bcat)


def _tc_layer3(agg, cntp, ycat, batch2d, W_lin, b_lin, interpret=False):
    n, dc = ycat.shape
    d = dc // 2
    out = W_lin.shape[1]
    R = 1000
    grid = (n // R,)
    last = grid[0] - 1

    def body(p_ref, c_ref, y_ref, b_ref, wlin, blin, out_ref, acc, accg):
        i = pl.program_id(0)

        @pl.when(i == 0)
        def _():
            acc[...] = jnp.zeros_like(acc)
            accg[...] = jnp.zeros_like(accg)

        psum = p_ref[0] + p_ref[1]
        cnt = jnp.maximum(c_ref[0] + c_ref[1], 1.0)
        mean = psum[:, :d] / cnt
        h3 = jnp.maximum(mean + y_ref[...][:, d:], 0.0)
        onehot = (b_ref[...] == lax.broadcasted_iota(jnp.int32, (R, _G), 1)
                  ).astype(jnp.float32)
        acc[...] += lax.dot_general(onehot, h3, (((0,), (0,)), ((), ())),
                                    preferred_element_type=jnp.float32)
        accg[...] += lax.dot_general(
            onehot, jnp.ones((R, 128), jnp.float32), (((0,), (0,)), ((), ())),
            preferred_element_type=jnp.float32)

        @pl.when(i == last)
        def _():
            pooled = acc[...] / jnp.maximum(accg[...][:, 0:1], 1.0)
            logits = _dot(pooled, wlin[...]) + blin[...]
            m = jnp.max(logits, axis=1, keepdims=True)
            lse = jnp.log(jnp.sum(jnp.exp(logits - m), axis=1, keepdims=True))
            out_ref[...] = logits - m - lse

    return pl.pallas_call(
        body,
        grid=grid,
        in_specs=[
            pl.BlockSpec((_NC, R, dc), lambda i: (0, i, 0)),
            pl.BlockSpec((_NC, R, 1), lambda i: (0, i, 0)),
            pl.BlockSpec((R, dc), lambda i: (i, 0)),
            pl.BlockSpec((R, 1), lambda i: (i, 0)),
            pl.BlockSpec((d, out), lambda i: (0, 0)),
            pl.BlockSpec((1, out), lambda i: (0, 0)),
        ],
        out_specs=pl.BlockSpec((_G, out), lambda i: (0, 0)),
        out_shape=jax.ShapeDtypeStruct((_G, out), jnp.float32),
        scratch_shapes=[
            pltpu.VMEM((_G, d), jnp.float32),
            pltpu.VMEM((_G, 128), jnp.float32),
        ],
        interpret=interpret,
    )(agg, cntp, ycat, batch2d, W_lin, b_lin.reshape(1, -1))


def kernel(x, edge_index, batch, W_l1, b_l1, W_r1, W_l2, b_l2, W_r2,
           W_l3, b_l3, W_r3, W_lin, b_lin):
    src = edge_index[0]
    dst = edge_index[1]
    cntp = _sc_count(dst, x.shape[0])
    agg1 = _sc_aggregate(x, src, dst)
    y2, s2 = _tc_layer1(agg1, cntp, x, W_l1, b_l1, W_r1, W_l2, b_l2, W_r2)
    agg2 = _sc_aggregate(y2, src, dst)
    Wcat = jnp.concatenate([W_l3, W_r3], axis=1)
    bcat = jnp.concatenate(
        [jnp.zeros_like(b_l3), b_l3]).reshape(1, -1)
    ycat = _tc_layer2(agg2, cntp, s2, Wcat, bcat)
    agg3 = _sc_aggregate(ycat, src, dst)
    return _tc_layer3(agg3, cntp, ycat, batch.reshape(-1, 1), W_lin, b_lin)


# speedup vs baseline: 1.0045x; 1.0045x over previous
import functools

import jax
import jax.numpy as jnp
from jax import lax
from jax.experimental import pallas as pl
from jax.experimental.pallas import tpu as pltpu
from jax.experimental.pallas import tpu_sc as plsc

_NC = 2
_NS = 16
_G = 32
_CW = 16


def _largest_divisor(n, cap):
    for c in range(cap, 0, -1):
        if n % c == 0 and c % 8 == 0:
            return c
    return None


def _sc_aggregate(y, src, dst):
    n, w = y.shape
    e = src.shape[0]
    nw = _NC * _NS
    assert e % nw == 0 and n % _NS == 0
    ept = e // nw
    ch = _largest_divisor(ept, 128)
    nchunk = ept // ch
    rows_pt = n // _NS
    nbuf = 3

    mesh = plsc.VectorSubcoreMesh(core_axis_name="c", subcore_axis_name="s")
    out_type = jax.ShapeDtypeStruct((_NC, _NS, rows_pt, w), jnp.float32)
    assert nchunk >= nbuf + 1
    scratch = (
        [pltpu.VMEM((ept,), jnp.int32),
         pltpu.VMEM((ept,), jnp.int32)]
        + [pltpu.VMEM((ch, w), jnp.float32) for _ in range(nbuf)]
        + [pltpu.VMEM_SHARED((n, w), jnp.float32)]
        + [pltpu.SemaphoreType.DMA for _ in range(nbuf)]
        + [pltpu.SemaphoreType.DMA,
           pltpu.SemaphoreType.DMA]
    )

    def body(y_hbm, src_hbm, dst_hbm, zer_hbm, acc_out, *rest):
        srcs_v, dsts_v = rest[:2]
        bufs = rest[2:2 + nbuf]
        acc_s = rest[2 + nbuf]
        semgs = rest[3 + nbuf:3 + 2 * nbuf]
        semi, semz = rest[3 + 2 * nbuf:5 + 2 * nbuf]
        cid = lax.axis_index("c")
        sid = lax.axis_index("s")
        wid = sid * _NC + cid

        ebase = wid * ept
        pltpu.async_copy(src_hbm.at[pl.ds(ebase, ept)], srcs_v, semi)
        pltpu.async_copy(dst_hbm.at[pl.ds(ebase, ept)], dsts_v, semi)

        r0 = sid * rows_pt
        zsrc = zer_hbm.at[sid]
        zdst = acc_s.at[pl.ds(r0, rows_pt)]
        pltpu.async_copy(zsrc, zdst, semz)
        pltpu.make_async_copy(zsrc, zdst, semz).wait()
        plsc.subcore_barrier()

        pltpu.make_async_copy(src_hbm.at[pl.ds(ebase, ept)], srcs_v,
                              semi).wait()
        pltpu.make_async_copy(dst_hbm.at[pl.ds(ebase, ept)], dsts_v,
                              semi).wait()

        def gath(j, buf, sem):
            pltpu.async_copy(y_hbm.at[srcs_v.at[pl.ds(j * ch, ch)]],
                             buf, sem)

        def gwait(j, buf, sem):
            pltpu.make_async_copy(y_hbm.at[srcs_v.at[pl.ds(j * ch, ch)]],
                                  buf, sem).wait()

        def scat(j, buf):
            pltpu.sync_copy(buf, acc_s.at[dsts_v.at[pl.ds(j * ch, ch)]],
                            add=True)

        for j in range(nbuf - 1):
            gath(j, bufs[j], semgs[j])

        def group(k, _):
            for i in range(nbuf):
                j = nbuf * k + i
                s2 = (i + nbuf - 1) % nbuf
                gwait(j, bufs[i], semgs[i])
                gath(j + nbuf - 1, bufs[s2], semgs[s2])
                scat(j, bufs[i])
            return 0

        ngrp = (nchunk - nbuf + 1) // nbuf
        lax.fori_loop(0, ngrp, group, 0)
        issued = nbuf * ngrp + nbuf - 2
        for j in range(nbuf * ngrp, nchunk):
            jg = j + nbuf - 1
            if jg < nchunk and jg > issued:
                gath(jg, bufs[jg % nbuf], semgs[jg % nbuf])
                issued = jg
            s = j % nbuf
            gwait(j, bufs[s], semgs[s])
            scat(j, bufs[s])
        plsc.subcore_barrier()

        pltpu.sync_copy(acc_s.at[pl.ds(r0, rows_pt)], acc_out.at[cid, sid])

    fn = pl.kernel(body, out_type=out_type, mesh=mesh, scratch_types=scratch)
    res = fn(y, src, dst, jnp.zeros((_NS, rows_pt, w), jnp.float32))
    return res.reshape(_NC, n, w)


def _sc_count(dst, n):
    e = dst.shape[0]
    nw = _NC * _NS
    assert e % nw == 0
    ept = e // nw
    ch = _largest_divisor(ept, 128)
    nchunk = ept // ch
    cpt = -(-n // (8 * _NS)) * 8
    ncpad = cpt * _NS
    zcn = -(-cpt // 16) * 16

    mesh = plsc.VectorSubcoreMesh(core_axis_name="c", subcore_axis_name="s")
    out_type = jax.ShapeDtypeStruct((_NC * ncpad,), jnp.float32)
    scratch = [
        pltpu.VMEM((ept,), jnp.int32),
        pltpu.VMEM((ch,), jnp.float32),
        pltpu.VMEM((zcn,), jnp.float32),
        pltpu.VMEM_SHARED((ncpad,), jnp.float32),
        pltpu.SemaphoreType.DMA,
    ]

    def body(dst_hbm, cnt_out, dsts_v, ones_v, zc_v, cnt_s, semi):
        cid = lax.axis_index("c")
        sid = lax.axis_index("s")
        wid = sid * _NC + cid
        ebase = wid * ept
        pltpu.async_copy(dst_hbm.at[pl.ds(ebase, ept)], dsts_v, semi)

        def fill16(ref, val):
            def go(j, _):
                ref[pl.ds(j * 16, 16)] = jnp.full((16,), val, jnp.float32)
                return 0
            lax.fori_loop(0, ref.shape[0] // 16, go, 0)

        fill16(ones_v, 1.0)
        fill16(zc_v, 0.0)
        pltpu.sync_copy(zc_v.at[pl.ds(0, cpt)],
                        cnt_s.at[pl.ds(sid * cpt, cpt)])
        pltpu.make_async_copy(dst_hbm.at[pl.ds(ebase, ept)], dsts_v,
                              semi).wait()
        plsc.subcore_barrier()

        def step(j, _):
            pltpu.sync_copy(ones_v,
                            cnt_s.at[dsts_v.at[pl.ds(j * ch, ch)]],
                            add=True)
            return 0

        lax.fori_loop(0, nchunk, step, 0)
        plsc.subcore_barrier()
        pltpu.sync_copy(cnt_s.at[pl.ds(sid * cpt, cpt)],
                        zc_v.at[pl.ds(0, cpt)])
        pltpu.sync_copy(zc_v.at[pl.ds(0, cpt)],
                        cnt_out.at[pl.ds(cid * ncpad + sid * cpt, cpt)])

    fn = pl.kernel(body, out_type=out_type, mesh=mesh, scratch_types=scratch)
    return fn(dst).reshape(_NC, ncpad)[:, :, None]


def _dot(a, b):
    return jnp.dot(a, b, preferred_element_type=jnp.float32)


def _mean_from_partials(p_ref, c_ref):
    psum = p_ref[0] + p_ref[1]
    cnt = jnp.maximum(c_ref[0] + c_ref[1], 1.0)
    return psum / cnt


def _tc_layer1(agg, cntp, x, W_l1, b_l1, W_r1, W_l2, b_l2, W_r2, interpret=False):
    n, d = x.shape
    k1 = W_l1.shape[1]
    k2 = W_l2.shape[1]
    R = 1000
    grid = (n // R,)

    def body(p_ref, c_ref, x_ref, wl1, bl1, wr1, wl2, bl2, wr2, y2_ref, s2_ref):
        mean = _mean_from_partials(p_ref, c_ref)
        h1 = jnp.maximum(
            _dot(mean, wl1[...]) + bl1[...] + _dot(x_ref[...], wr1[...]), 0.0)
        y2_ref[...] = _dot(h1, wl2[...])
        s2_ref[...] = _dot(h1, wr2[...]) + bl2[...]

    return pl.pallas_call(
        body,
        grid=grid,
        in_specs=[
            pl.BlockSpec((_NC, R, d), lambda i: (0, i, 0)),
            pl.BlockSpec((_NC, R, 1), lambda i: (0, i, 0)),
            pl.BlockSpec((R, d), lambda i: (i, 0)),
            pl.BlockSpec((d, k1), lambda i: (0, 0)),
            pl.BlockSpec((1, k1), lambda i: (0, 0)),
            pl.BlockSpec((d, k1), lambda i: (0, 0)),
            pl.BlockSpec((k1, k2), lambda i: (0, 0)),
            pl.BlockSpec((1, k2), lambda i: (0, 0)),
            pl.BlockSpec((k1, k2), lambda i: (0, 0)),
        ],
        out_specs=[
            pl.BlockSpec((R, k2), lambda i: (i, 0)),
            pl.BlockSpec((R, k2), lambda i: (i, 0)),
        ],
        out_shape=[
            jax.ShapeDtypeStruct((n, k2), jnp.float32),
            jax.ShapeDtypeStruct((n, k2), jnp.float32),
        ],
        interpret=interpret,
    )(agg, cntp, x, W_l1, b_l1.reshape(1, -1), W_r1, W_l2,
      b_l2.reshape(1, -1), W_r2)


def _tc_layer2(agg, cntp, s2, Wcat, bcat, interpret=False):
    n, d = s2.shape
    k2 = Wcat.shape[1]
    R = 1000
    grid = (n // R,)

    def body(p_ref, c_ref, s2_ref, wcat, bc, ycat_ref):
        mean = _mean_from_partials(p_ref, c_ref)
        h2 = jnp.maximum(mean + s2_ref[...], 0.0)
        ycat_ref[...] = _dot(h2, wcat[...]) + bc[...]

    return pl.pallas_call(
        body,
        grid=grid,
        in_specs=[
            pl.BlockSpec((_NC, R, d), lambda i: (0, i, 0)),
            pl.BlockSpec((_NC, R, 1), lambda i: (0, i, 0)),
            pl.BlockSpec((R, d), lambda i: (i, 0)),
            pl.BlockSpec((d, k2), lambda i: (0, 0)),
            pl.BlockSpec((1, k2), lambda i: (0, 0)),
        ],
        out_specs=pl.BlockSpec((R, k2), lambda i: (i, 0)),
        out_shape=jax.ShapeDtypeStruct((n, k2), jnp.float32),
        interpret=interpret,
    )(agg, cntp, s2, Wcat, bcat)


def _tc_layer3(agg, cntp, ycat, batch2d, W_lin, b_lin, interpret=False):
    n, dc = ycat.shape
    d = dc // 2
    out = W_lin.shape[1]
    R = 1000
    grid = (n // R,)
    last = grid[0] - 1

    def body(p_ref, c_ref, y_ref, b_ref, wlin, blin, out_ref, acc, accg):
        i = pl.program_id(0)

        @pl.when(i == 0)
        def _():
            acc[...] = jnp.zeros_like(acc)
            accg[...] = jnp.zeros_like(accg)

        psum = p_ref[0] + p_ref[1]
        cnt = jnp.maximum(c_ref[0] + c_ref[1], 1.0)
        mean = psum[:, :d] / cnt
        h3 = jnp.maximum(mean + y_ref[...][:, d:], 0.0)
        onehot = (b_ref[...] == lax.broadcasted_iota(jnp.int32, (R, _G), 1)
                  ).astype(jnp.float32)
        acc[...] += lax.dot_general(onehot, h3, (((0,), (0,)), ((), ())),
                                    preferred_element_type=jnp.float32)
        accg[...] += lax.dot_general(
            onehot, jnp.ones((R, 128), jnp.float32), (((0,), (0,)), ((), ())),
            preferred_element_type=jnp.float32)

        @pl.when(i == last)
        def _():
            pooled = acc[...] / jnp.maximum(accg[...][:, 0:1], 1.0)
            logits = _dot(pooled, wlin[...]) + blin[...]
            m = jnp.max(logits, axis=1, keepdims=True)
            lse = jnp.log(jnp.sum(jnp.exp(logits - m), axis=1, keepdims=True))
            out_ref[...] = logits - m - lse

    return pl.pallas_call(
        body,
        grid=grid,
        in_specs=[
            pl.BlockSpec((_NC, R, dc), lambda i: (0, i, 0)),
            pl.BlockSpec((_NC, R, 1), lambda i: (0, i, 0)),
            pl.BlockSpec((R, dc), lambda i: (i, 0)),
            pl.BlockSpec((R, 1), lambda i: (i, 0)),
            pl.BlockSpec((d, out), lambda i: (0, 0)),
            pl.BlockSpec((1, out), lambda i: (0, 0)),
        ],
        out_specs=pl.BlockSpec((_G, out), lambda i: (0, 0)),
        out_shape=jax.ShapeDtypeStruct((_G, out), jnp.float32),
        scratch_shapes=[
            pltpu.VMEM((_G, d), jnp.float32),
            pltpu.VMEM((_G, 128), jnp.float32),
        ],
        interpret=interpret,
    )(agg, cntp, ycat, batch2d, W_lin, b_lin.reshape(1, -1))


def kernel(x, edge_index, batch, W_l1, b_l1, W_r1, W_l2, b_l2, W_r2,
           W_l3, b_l3, W_r3, W_lin, b_lin):
    src = edge_index[0]
    dst = edge_index[1]
    cntp = _sc_count(dst, x.shape[0])
    agg1 = _sc_aggregate(x, src, dst)
    y2, s2 = _tc_layer1(agg1, cntp, x, W_l1, b_l1, W_r1, W_l2, b_l2, W_r2)
    agg2 = _sc_aggregate(y2, src, dst)
    Wcat = jnp.concatenate([W_l3, W_r3], axis=1)
    bcat = jnp.concatenate(
        [jnp.zeros_like(b_l3), b_l3]).reshape(1, -1)
    ycat = _tc_layer2(agg2, cntp, s2, Wcat, bcat)
    agg3 = _sc_aggregate(ycat, src, dst)
    return _tc_layer3(agg3, cntp, ycat, batch.reshape(-1, 1), W_lin, b_lin)
